# Initial kernel scaffold; baseline (speedup 1.0000x reference)
import jax
import jax.numpy as jnp
from jax.experimental import pallas as pl


def kernel(node_type, velocity, cells, mesh_pos, is_training, params):
    raise NotImplementedError("write your pallas kernel here")


# speedup vs baseline: 1.4997x; 1.4997x over previous
import functools

import jax
import jax.numpy as jnp
from jax import lax
from jax.experimental import pallas as pl
from jax.experimental.pallas import tpu as pltpu
from jax.experimental.pallas import tpu_sc as plsc

N = 10000
NPAD = 10240
TRASH = 10000
M = 60000
MPAD = 65536
NROWS = 512
NN = NPAD * NPAD
SENT = NN
LATENT = 128
NC, NS = 2, 16
NW = NC * NS
RW = NROWS // NW
EB = RW * 128
RB = 1024
NTILE = NPAD // NS
TS = NROWS // NS


def _mesh():
    return plsc.VectorSubcoreMesh(core_axis_name="c", subcore_axis_name="s")


def _d1_body(keys_hbm, vals_hbm, t_hbm, keys_v, vals_v, sem):
    c = lax.axis_index("c")
    s = lax.axis_index("s")
    base = (s * NC + c) * RW
    pltpu.sync_copy(keys_hbm.at[pl.ds(base, RW)], keys_v)
    pltpu.sync_copy(vals_hbm.at[pl.ds(base, RW)], vals_v)
    descs = []
    for j in range(RW):
        descs.append(pltpu.async_copy(vals_v.at[j], t_hbm.at[keys_v.at[j]], sem))
    for d in descs:
        d.wait()


def _dedup_scatter(keys2d, vals2d):
    f = pl.kernel(
        _d1_body,
        out_type=jax.ShapeDtypeStruct((NN + 64,), jnp.int32),
        mesh=_mesh(),
        scratch_types=[
            pltpu.VMEM((RW, 128), jnp.int32),
            pltpu.VMEM((RW, 128), jnp.int32),
            pltpu.SemaphoreType.DMA,
        ],
    )
    return f(keys2d, vals2d)


def _d2_body(t_hbm, keys_hbm, vals_hbm, lo_hbm, hi_hbm,
             keep_hbm, s1_hbm, s2_hbm,
             keys_v, vals_v, lo_v, hi_v, back_v, keep_v, s1_v, s2_v, sema):
    c = lax.axis_index("c")
    s = lax.axis_index("s")
    base = (s * NC + c) * RW
    pltpu.sync_copy(keys_hbm.at[pl.ds(base, RW)], keys_v)
    pltpu.sync_copy(vals_hbm.at[pl.ds(base, RW)], vals_v)
    pltpu.sync_copy(lo_hbm.at[pl.ds(base, RW)], lo_v)
    pltpu.sync_copy(hi_hbm.at[pl.ds(base, RW)], hi_v)
    descs = []
    for j in range(RW):
        descs.append(pltpu.async_copy(t_hbm.at[keys_v.at[j]], back_v.at[j], sema))
    for d in descs:
        d.wait()
    for j in range(RW):
        for k in range(8):
            sl = pl.ds(k * 16, 16)
            t16 = back_v[j, sl]
            v16 = vals_v[j, sl]
            m = (t16 == v16) & (v16 < M)
            keep_v[j, sl] = jnp.where(m, 1.0, 0.0)
            s1_v[j, sl] = jnp.where(m, hi_v[j, sl], TRASH)
            s2_v[j, sl] = jnp.where(m, lo_v[j, sl], TRASH)
    pltpu.sync_copy(keep_v, keep_hbm.at[pl.ds(base, RW)])
    pltpu.sync_copy(s1_v, s1_hbm.at[pl.ds(base, RW)])
    pltpu.sync_copy(s2_v, s2_hbm.at[pl.ds(base, RW)])


def _dedup_gather(t, keys2d, vals2d, lo2d, hi2d):
    f = pl.kernel(
        _d2_body,
        out_type=[
            jax.ShapeDtypeStruct((NROWS, 128), jnp.float32),
            jax.ShapeDtypeStruct((NROWS, 128), jnp.int32),
            jax.ShapeDtypeStruct((NROWS, 128), jnp.int32),
        ],
        mesh=_mesh(),
        scratch_types=[
            pltpu.VMEM((RW, 128), jnp.int32),
            pltpu.VMEM((RW, 128), jnp.int32),
            pltpu.VMEM((RW, 128), jnp.int32),
            pltpu.VMEM((RW, 128), jnp.int32),
            pltpu.VMEM((RW, 128), jnp.int32),
            pltpu.VMEM((RW, 128), jnp.float32),
            pltpu.VMEM((RW, 128), jnp.int32),
            pltpu.VMEM((RW, 128), jnp.int32),
            pltpu.SemaphoreType.DMA,
        ],
    )
    return f(t, keys2d, vals2d, lo2d, hi2d)


def _g_body(lat_hbm, lo_hbm, hi_hbm, glo_hbm, ghi_hbm,
            lo_v, hi_v, abuf, bbuf, gs0, gs1, ss0, ss1):
    c = lax.axis_index("c")
    s = lax.axis_index("s")
    wid = s * NC + c
    base = wid * RW
    ebase = wid * EB
    gsem = (gs0, gs1)
    ssem = (ss0, ss1)
    pltpu.sync_copy(lo_hbm.at[pl.ds(base, RW)], lo_v)
    pltpu.sync_copy(hi_hbm.at[pl.ds(base, RW)], hi_v)
    gd = [None] * RW
    sd = [None] * RW
    for j in range(RW + 1):
        sl = j & 1
        if j < RW:
            if j >= 2:
                for d in sd[j - 2]:
                    d.wait()
            gd[j] = (
                pltpu.async_copy(lat_hbm.at[lo_v.at[j]], abuf.at[sl], gsem[sl]),
                pltpu.async_copy(lat_hbm.at[hi_v.at[j]], bbuf.at[sl], gsem[sl]),
            )
        if j >= 1:
            for d in gd[j - 1]:
                d.wait()
            osl = (j - 1) & 1
            sd[j - 1] = (
                pltpu.async_copy(abuf.at[osl],
                                 glo_hbm.at[pl.ds(ebase + (j - 1) * 128, 128)],
                                 ssem[osl]),
                pltpu.async_copy(bbuf.at[osl],
                                 ghi_hbm.at[pl.ds(ebase + (j - 1) * 128, 128)],
                                 ssem[osl]),
            )
    for d in sd[RW - 2]:
        d.wait()
    for d in sd[RW - 1]:
        d.wait()


def _gather_lat(lat, lo2d, hi2d):
    f = pl.kernel(
        _g_body,
        out_type=[
            jax.ShapeDtypeStruct((MPAD, LATENT), jnp.float32),
            jax.ShapeDtypeStruct((MPAD, LATENT), jnp.float32),
        ],
        mesh=_mesh(),
        scratch_types=[
            pltpu.VMEM((RW, 128), jnp.int32),
            pltpu.VMEM((RW, 128), jnp.int32),
            pltpu.VMEM((2, 128, LATENT), jnp.float32),
            pltpu.VMEM((2, 128, LATENT), jnp.float32),
            pltpu.SemaphoreType.DMA,
            pltpu.SemaphoreType.DMA,
            pltpu.SemaphoreType.DMA,
            pltpu.SemaphoreType.DMA,
        ],
    )
    return f(lat, lo2d, hi2d)


def _s_body(e1_hbm, e2_hbm, s1_hbm, s2_hbm, z_hbm, parts_hbm,
            acc, ebuf, idx_v, gsem):
    c = lax.axis_index("c")
    s = lax.axis_index("s")
    for k in range(NTILE // 128):
        pltpu.sync_copy(z_hbm, acc.at[pl.ds(s * NTILE + k * 128, 128)])

    @pl.when(c == 0)
    def _():
        pltpu.sync_copy(s1_hbm.at[pl.ds(s * TS, TS)], idx_v)

    @pl.when(c != 0)
    def _():
        pltpu.sync_copy(s2_hbm.at[pl.ds(s * TS, TS)], idx_v)

    plsc.subcore_barrier()

    def scatter_half(e_hbm):
        gd = [None] * TS
        for j in range(TS + 1):
            sl = j & 1
            if j < TS:
                gd[j] = pltpu.async_copy(
                    e_hbm.at[pl.ds((s * TS + j) * 128, 128)], ebuf.at[sl], gsem)
            if j >= 1:
                gd[j - 1].wait()
                pltpu.sync_copy(ebuf.at[(j - 1) & 1], acc.at[idx_v.at[j - 1]],
                                add=True)

    @pl.when(c == 0)
    def _():
        scatter_half(e1_hbm)

    @pl.when(c != 0)
    def _():
        scatter_half(e2_hbm)

    plsc.subcore_barrier()
    for k in range(NTILE // 128):
        r = s * NTILE + k * 128
        pltpu.sync_copy(acc.at[pl.ds(r, 128)], parts_hbm.at[c, pl.ds(r, 128)])


def _scatter_add(e1, e2, s1_2d, s2_2d, zeros128):
    f = pl.kernel(
        _s_body,
        out_type=jax.ShapeDtypeStruct((2, NPAD, LATENT), jnp.float32),
        mesh=_mesh(),
        scratch_types=[
            pltpu.VMEM_SHARED((NPAD, LATENT), jnp.float32),
            pltpu.VMEM((2, 128, LATENT), jnp.float32),
            pltpu.VMEM((TS, 128), jnp.int32),
            pltpu.SemaphoreType.DMA,
        ],
    )
    return f(e1, e2, s1_2d, s2_2d, zeros128)


def _mm(x, w):
    return jnp.dot(x, w, preferred_element_type=jnp.float32)


def _relu(x):
    return jnp.maximum(x, 0.0)


def _ln(x, g, b):
    mu = jnp.mean(x, axis=-1, keepdims=True)
    xc = x - mu
    var = jnp.mean(xc * xc, axis=-1, keepdims=True)
    return xc * lax.rsqrt(var + 1e-5) * g + b


def _nenc_body(nf, w1, b1, w2, b2, w3, b3, g, bb, out):
    x = nf[...]
    mean = jnp.sum(x, axis=0, keepdims=True) * (1.0 / N)
    sq = jnp.sum(x * x, axis=0, keepdims=True) * (1.0 / N)
    std = jnp.maximum(jnp.sqrt(jnp.maximum(sq - mean * mean, 0.0)), 1e-8)
    xn = (x - mean) / std
    h = _relu(_mm(xn, w1[...]) + b1[...])
    h = _relu(_mm(h, w2[...]) + b2[...])
    h = _mm(h, w3[...]) + b3[...]
    out[...] = _ln(h, g[...], bb[...])


def _node_encode(nf, p):
    w1 = jnp.zeros((16, LATENT), jnp.float32).at[:11].set(p['W'][0])
    args = (nf, w1, p['b'][0][None], p['W'][1], p['b'][1][None],
            p['W'][2], p['b'][2][None], p['ln_g'][None], p['ln_b'][None])
    return pl.pallas_call(
        _nenc_body,
        out_shape=jax.ShapeDtypeStruct((NPAD, LATENT), jnp.float32),
    )(*args)


def _estat_body(plo, phi, keep, out):
    @pl.when(pl.program_id(0) == 0)
    def _():
        out[...] = jnp.zeros((8, 128), jnp.float32)

    d0 = plo[:, 0:1] - phi[:, 0:1]
    d1 = plo[:, 1:2] - phi[:, 1:2]
    nrm = jnp.sqrt(d0 * d0 + d1 * d1)
    k = keep[...]
    cnt = jnp.sum(k)
    sum_n = jnp.sum(nrm * k)
    s0 = jnp.sum(d0 * d0 * k)
    s1 = jnp.sum(d1 * d1 * k)
    sn = jnp.sum(nrm * nrm * k)
    lane = lax.broadcasted_iota(jnp.int32, (1, 128), 1)
    row = jnp.where(lane == 0, cnt,
                    jnp.where(lane == 1, sum_n,
                              jnp.where(lane == 2, s0,
                                        jnp.where(lane == 3, s1,
                                                  jnp.where(lane == 4, sn,
                                                            0.0)))))
    out[...] += jnp.concatenate([row, jnp.zeros((7, 128), jnp.float32)], 0)


def _edge_stats(plo, phi, keep1):
    espec = pl.BlockSpec((RB, LATENT), lambda i: (i, 0))
    return pl.pallas_call(
        _estat_body,
        grid=(MPAD // RB,),
        in_specs=[espec, espec, pl.BlockSpec((RB, 1), lambda i: (i, 0))],
        out_specs=pl.BlockSpec((8, 128), lambda i: (0, 0)),
        out_shape=jax.ShapeDtypeStruct((8, 128), jnp.float32),
    )(plo, phi, keep1)


def _eenc_body(plo, phi, w1r, b1, w2, b2, w3, b3, g, bb, o1, o2):
    d0 = plo[:, 0:1] - phi[:, 0:1]
    d1 = plo[:, 1:2] - phi[:, 1:2]
    nrm = jnp.sqrt(d0 * d0 + d1 * d1)
    t = d0 * w1r[0:1, :] + d1 * w1r[1:2, :]
    u = nrm * w1r[2:3, :] + b1[...]
    for h, o in ((_relu(u + t), o1), (_relu(u - t), o2)):
        h = _relu(_mm(h, w2[...]) + b2[...])
        h = _mm(h, w3[...]) + b3[...]
        o[...] = _ln(h, g[...], bb[...])


def _edge_encode(plo, phi, w1f, b1f, p):
    wspec = pl.BlockSpec((8, 128), lambda i: (0, 0))
    bspec = pl.BlockSpec((1, 128), lambda i: (0, 0))
    mspec = pl.BlockSpec((LATENT, LATENT), lambda i: (0, 0))
    espec = pl.BlockSpec((RB, LATENT), lambda i: (i, 0))
    w1r = jnp.zeros((8, 128), jnp.float32).at[:3].set(w1f)
    return pl.pallas_call(
        _eenc_body,
        grid=(MPAD // RB,),
        in_specs=[espec, espec,
                  wspec, bspec, mspec, bspec, mspec, bspec, bspec, bspec],
        out_specs=[espec, espec],
        out_shape=[jax.ShapeDtypeStruct((MPAD, LATENT), jnp.float32),
                   jax.ShapeDtypeStruct((MPAD, LATENT), jnp.float32)],
    )(plo, phi, w1r, b1f[None], p['W'][1], p['b'][1][None],
      p['W'][2], p['b'][2][None], p['ln_g'][None], p['ln_b'][None])


def _estep_body(e1, e2, glo, ghi, wa, wb, wc, b1, w2, b2, w3, b3, g, bb,
                o1, o2):
    x1 = e1[...]
    x2 = e2[...]
    gl = glo[...]
    gh = ghi[...]
    lb = _mm(gl, wb[...])
    hb = _mm(gh, wb[...])
    lc = _mm(gl, wc[...])
    hc = _mm(gh, wc[...])
    h1 = _relu(_mm(x1, wa[...]) + lb + hc + b1[...])
    h2 = _relu(_mm(x2, wa[...]) + hb + lc + b1[...])
    for h, x, o in ((h1, x1, o1), (h2, x2, o2)):
        h = _relu(_mm(h, w2[...]) + b2[...])
        h = _mm(h, w3[...]) + b3[...]
        o[...] = x + _ln(h, g[...], bb[...])


def _edge_step(e1, e2, glo, ghi, p):
    w1 = p['W'][0]
    bspec = pl.BlockSpec((1, 128), lambda i: (0, 0))
    mspec = pl.BlockSpec((LATENT, LATENT), lambda i: (0, 0))
    espec = pl.BlockSpec((RB, LATENT), lambda i: (i, 0))
    return pl.pallas_call(
        _estep_body,
        grid=(MPAD // RB,),
        in_specs=[espec, espec, espec, espec,
                  mspec, mspec, mspec, bspec, mspec, bspec, mspec, bspec,
                  bspec, bspec],
        out_specs=[espec, espec],
        out_shape=[jax.ShapeDtypeStruct((MPAD, LATENT), jnp.float32),
                   jax.ShapeDtypeStruct((MPAD, LATENT), jnp.float32)],
    )(e1, e2, glo, ghi, w1[:LATENT], w1[LATENT:2 * LATENT], w1[2 * LATENT:],
      p['b'][0][None], p['W'][1], p['b'][1][None], p['W'][2], p['b'][2][None],
      p['ln_g'][None], p['ln_b'][None])


def _nstep_body(lat, parts, w1t, w1b, b1, w2, b2, w3, b3, g, bb, out):
    x = lat[...]
    agg = parts[0] + parts[1]
    h = _relu(_mm(x, w1t[...]) + _mm(agg, w1b[...]) + b1[...])
    h = _relu(_mm(h, w2[...]) + b2[...])
    h = _mm(h, w3[...]) + b3[...]
    out[...] = x + _ln(h, g[...], bb[...])


def _node_step(lat, parts, p):
    w1 = p['W'][0]
    return pl.pallas_call(
        _nstep_body,
        out_shape=jax.ShapeDtypeStruct((NPAD, LATENT), jnp.float32),
    )(lat, parts, w1[:LATENT], w1[LATENT:], p['b'][0][None],
      p['W'][1], p['b'][1][None], p['W'][2], p['b'][2][None],
      p['ln_g'][None], p['ln_b'][None])


def _dec_body(lat, w1, b1, w2, b2, w3, b3, out):
    h = _relu(_mm(lat[...], w1[...]) + b1[...])
    h = _relu(_mm(h, w2[...]) + b2[...])
    out[...] = _mm(h, w3[...]) + b3[...]


def _decode(lat, p):
    w3 = jnp.zeros((LATENT, 128), jnp.float32).at[:, :2].set(p['W'][2])
    b3 = jnp.zeros((128,), jnp.float32).at[:2].set(p['b'][2])
    return pl.pallas_call(
        _dec_body,
        out_shape=jax.ShapeDtypeStruct((NPAD, 128), jnp.float32),
    )(lat, p['W'][0], p['b'][0][None], p['W'][1], p['b'][1][None],
      w3, b3[None])


def kernel(node_type, velocity, cells, mesh_pos, is_training, params):
    cells = cells.astype(jnp.int32)
    a, b, c = cells[:, 0], cells[:, 1], cells[:, 2]
    e = jnp.concatenate([jnp.stack([a, b], 1), jnp.stack([b, c], 1),
                         jnp.stack([c, a], 1)], axis=0)
    lo = jnp.minimum(e[:, 0], e[:, 1])
    hi = jnp.maximum(e[:, 0], e[:, 1])
    pad = jnp.zeros((MPAD - M,), jnp.int32)
    lo = jnp.concatenate([lo, pad])
    hi = jnp.concatenate([hi, pad])
    keys = jnp.where(jnp.arange(MPAD) < M, lo * NPAD + hi, SENT)
    vals = jnp.arange(MPAD, dtype=jnp.int32)
    keys2d = keys.astype(jnp.int32).reshape(NROWS, 128)
    vals2d = vals.reshape(NROWS, 128)
    lo2d = lo.reshape(NROWS, 128)
    hi2d = hi.reshape(NROWS, 128)
    pos128 = jnp.zeros((NPAD, 128), jnp.float32).at[:N, :2].set(mesh_pos)

    t = _dedup_scatter(keys2d, vals2d)
    keepf2d, s1_2d, s2_2d = _dedup_gather(t, keys2d, vals2d, lo2d, hi2d)
    plo, phi = _gather_lat(pos128, lo2d, hi2d)
    keep1 = keepf2d.reshape(MPAD, 1)

    one_hot = jax.nn.one_hot(node_type[:, 0], 9, dtype=jnp.float32)
    nf = jnp.zeros((NPAD, 16), jnp.float32).at[:N, :11].set(
        jnp.concatenate([velocity, one_hot], axis=-1))
    lat = _node_encode(nf, params['node_encoder'])

    stats = _edge_stats(plo, phi, keep1)
    cnt = stats[0, 0]
    mean_n = stats[0, 1] / cnt
    sq = stats[0, 2:5] / cnt
    mean_e = jnp.array([0.0, 0.0, 1.0], jnp.float32) * mean_n
    std_e = jnp.maximum(jnp.sqrt(jnp.maximum(sq - mean_e * mean_e, 0.0)), 1e-8)
    rstd_e = 1.0 / std_e
    pe = params['edge_encoder']
    w1f = pe['W'][0] * rstd_e[:, None]
    b1f = pe['b'][0] - (mean_e * rstd_e) @ pe['W'][0]
    e1, e2 = _edge_encode(plo, phi, w1f, b1f, pe)

    zeros128 = jnp.zeros((128, LATENT), jnp.float32)
    for sp in params['steps']:
        glo, ghi = _gather_lat(lat, lo2d, hi2d)
        e1, e2 = _edge_step(e1, e2, glo, ghi, sp['edge'])
        parts = _scatter_add(e1, e2, s1_2d, s2_2d, zeros128)
        lat = _node_step(lat, parts, sp['node'])

    out = _decode(lat, params['decoder'])
    return out[:N, :2]
